# Initial kernel scaffold; baseline (speedup 1.0000x reference)
import jax
import jax.numpy as jnp
from jax.experimental import pallas as pl


def kernel(x, W1, b1, W2, batch):
    raise NotImplementedError("write your pallas kernel here")


# speedup vs baseline: 6.2133x; 6.2133x over previous
import math

import jax
import jax.numpy as jnp
from jax.experimental import pallas as pl
from jax.experimental.pallas import tpu as pltpu

S = 256

_HIGH = jax.lax.Precision.HIGHEST


def _body(x_ref, batch_ref, W1_ref, b1_ref, W2r_ref, outT_ref, m_ref, d_ref):
    pid = pl.program_id(0)
    nb = pl.num_programs(0)

    @pl.when(pid == 0)
    def _init():
        outT_ref[...] = jnp.zeros_like(outT_ref)
        m_ref[...] = jnp.full(m_ref.shape, -jnp.inf, jnp.float32)
        d_ref[...] = jnp.zeros_like(d_ref)

    x = x_ref[...]
    R = x.shape[0]
    h = jnp.tanh(
        jax.lax.dot(x, W1_ref[...], precision=_HIGH) + b1_ref[...]
    )
    s = jax.lax.dot_general(
        h, W2r_ref[...], (((1,), (1,)), ((), ())), precision=_HIGH
    )

    ids = batch_ref[...].reshape(R, 1)
    seg = jax.lax.broadcasted_iota(jnp.int32, (R, S), 1)
    mask = ids == seg
    neg = jnp.float32(-jnp.inf)

    m_blk = jnp.max(jnp.where(mask, s, neg), axis=0, keepdims=True)
    m_old = m_ref[...]
    m_new = jnp.maximum(m_old, m_blk)
    scale = jnp.where(m_new == neg, 0.0, jnp.exp(m_old - m_new))
    m_ref[...] = m_new

    m_row = jnp.max(jnp.where(mask, m_new, neg), axis=1, keepdims=True)
    e = jnp.where(m_row == neg, 0.0, jnp.exp(s - m_row))

    maskf = mask.astype(jnp.float32)
    d_blk = jnp.sum(maskf * e, axis=0, keepdims=True)
    d_ref[...] = d_ref[...] * scale + d_blk

    xe = x * e
    accT = jax.lax.dot_general(
        xe, maskf, (((0,), (0,)), ((), ())), precision=_HIGH
    )
    outT_ref[...] = outT_ref[...] * scale + accT

    @pl.when(pid == nb - 1)
    def _finish():
        outT_ref[...] = outT_ref[...] / (d_ref[...] + 1e-16)


def _pick_block(n: int) -> int:
    for r in range(2048, 7, -8):
        if n % r == 0:
            return r
    return 0


def kernel(x, W1, b1, W2, batch):
    N, D = x.shape
    R = _pick_block(N)
    if R == 0:
        R = 2048
        pad = (-N) % R
        x = jnp.pad(x, ((0, pad), (0, 0)))
        batch = jnp.concatenate(
            [batch, jnp.full((pad,), S, dtype=batch.dtype)]
        )
        N = N + pad
    NB = N // R

    batch3 = batch.astype(jnp.int32).reshape(NB, R, 1)
    b1r = b1.reshape(1, D)
    W2r = W2.reshape(1, D)

    outT = pl.pallas_call(
        _body,
        grid=(NB,),
        in_specs=[
            pl.BlockSpec((R, D), lambda i: (i, 0)),
            pl.BlockSpec((1, R, 1), lambda i: (i, 0, 0)),
            pl.BlockSpec((D, D), lambda i: (0, 0)),
            pl.BlockSpec((1, D), lambda i: (0, 0)),
            pl.BlockSpec((1, D), lambda i: (0, 0)),
        ],
        out_specs=pl.BlockSpec((D, S), lambda i: (0, 0)),
        out_shape=jax.ShapeDtypeStruct((D, S), jnp.float32),
        scratch_shapes=[
            pltpu.VMEM((1, S), jnp.float32),
            pltpu.VMEM((1, S), jnp.float32),
        ],
    )(x, batch3, W1, b1r, W2r)
    return outT.T
